# baseline (speedup 1.0000x reference)
import functools

import jax
import jax.numpy as jnp
from jax import lax
from jax.experimental import pallas as pl
from jax.experimental.pallas import tpu as pltpu
from jax.experimental.pallas import tpu_sc as plsc

N = 10000
E = 160000
HZ = 512
NL = 3

NCH = 4
CHW = 128
NSC = 2
NTS = 16
NW = NSC * NTS

A_EPT = 5120
E_PAD = NW * A_EPT
A_CH = 1024
A_NCH = A_EPT // A_CH

B_EPT = E // NTS
B_K = 80
B_NB = B_EPT // B_K
ROWS_PT = N // NTS
RP = 125

MB = 1000
GM = N // MB

_f32 = jnp.float32


def _mm_bias_body(x_ref, w_ref, b_ref, o_ref):
    o_ref[...] = jnp.dot(x_ref[...], w_ref[...],
                         preferred_element_type=_f32) + b_ref[...]


def _mm_bias(x, w, b, mb=MB):
    m, k = x.shape
    n = w.shape[1]
    return pl.pallas_call(
        _mm_bias_body,
        grid=(m // mb,),
        in_specs=[
            pl.BlockSpec((mb, k), lambda i: (i, 0)),
            pl.BlockSpec((k, n), lambda i: (0, 0)),
            pl.BlockSpec((1, n), lambda i: (0, 0)),
        ],
        out_specs=pl.BlockSpec((mb, n), lambda i: (i, 0)),
        out_shape=jax.ShapeDtypeStruct((m, n), _f32),
    )(x, w, b.reshape(1, n))


def _mmz_body(h_ref, wc_ref, bc_ref, z_ref):
    z_ref[...] = jnp.dot(h_ref[...], wc_ref[...],
                         preferred_element_type=_f32) + bc_ref[...]


def _mmz(h, wc, bc):
    return pl.pallas_call(
        _mmz_body,
        grid=(GM, NCH),
        in_specs=[
            pl.BlockSpec((MB, HZ), lambda m, c: (m, 0)),
            pl.BlockSpec((HZ, CHW), lambda m, c: (0, c)),
            pl.BlockSpec((1, CHW), lambda m, c: (0, c)),
        ],
        out_specs=pl.BlockSpec((MB, CHW), lambda m, c: (c * GM + m, 0)),
        out_shape=jax.ShapeDtypeStruct((NCH * N, CHW), _f32),
    )(h, wc, bc.reshape(1, HZ))


def _mmszd_body(h_ref, u_ref, cu_ref, szd_ref):
    szd_ref[...] = jnp.dot(h_ref[...], u_ref[...],
                           preferred_element_type=_f32) + cu_ref[...]


def _mmszd(h, u_pad, cu_pad):
    return pl.pallas_call(
        _mmszd_body,
        grid=(GM,),
        in_specs=[
            pl.BlockSpec((MB, HZ), lambda m: (m, 0)),
            pl.BlockSpec((HZ, 128), lambda m: (0, 0)),
            pl.BlockSpec((1, 128), lambda m: (0, 0)),
        ],
        out_specs=pl.BlockSpec((MB, 128), lambda m: (m, 0)),
        out_shape=jax.ShapeDtypeStruct((N, 128), _f32),
    )(h, u_pad, cu_pad)


def _mm2_body(hc_ref, den_ref, w1_ref, b1_ref, y_ref, st_ref):
    m = pl.program_id(0)
    kc = pl.program_id(1)
    recip = 1.0 / (den_ref[:, 0:1] + 1e-16)
    a = hc_ref[...] * recip
    part = jnp.dot(a, w1_ref[...], preferred_element_type=_f32)

    @pl.when(kc == 0)
    def _():
        y_ref[...] = part + b1_ref[...]

    @pl.when(kc > 0)
    def _():
        y_ref[...] += part

    @pl.when(kc == NCH - 1)
    def _():
        y = y_ref[...]
        s = jnp.sum(y, axis=0, keepdims=True)
        s2 = jnp.sum(y * y, axis=0, keepdims=True)
        st = jnp.concatenate([s, s2, jnp.zeros((6, y.shape[1]), _f32)], axis=0)

        @pl.when(m == 0)
        def _():
            st_ref[...] = st

        @pl.when(m > 0)
        def _():
            st_ref[...] += st


def _mm2(hconv, den2d, w1, b1):
    h2 = 2 * HZ
    return pl.pallas_call(
        _mm2_body,
        grid=(GM, NCH),
        in_specs=[
            pl.BlockSpec((MB, CHW), lambda m, kc: (kc * GM + m, 0)),
            pl.BlockSpec((MB, 16), lambda m, kc: (m, 0)),
            pl.BlockSpec((CHW, h2), lambda m, kc: (kc, 0)),
            pl.BlockSpec((1, h2), lambda m, kc: (0, 0)),
        ],
        out_specs=[
            pl.BlockSpec((MB, h2), lambda m, kc: (m, 0)),
            pl.BlockSpec((8, h2), lambda m, kc: (0, 0)),
        ],
        out_shape=[
            jax.ShapeDtypeStruct((N, h2), _f32),
            jax.ShapeDtypeStruct((8, h2), _f32),
        ],
    )(hconv, den2d, w1, b1.reshape(1, h2))


def _mm3_body(y_ref, st_ref, g_ref, be_ref, w2_ref, b2_ref, hp_ref, o_ref):
    mu = st_ref[0:1, :] * (1.0 / N)
    var = st_ref[1:2, :] * (1.0 / N) - mu * mu
    aa = g_ref[...] * lax.rsqrt(var + 1e-5)
    bb = be_ref[...] - mu * aa
    t = y_ref[...] * aa + bb
    t = jnp.maximum(t, 0.01 * t)
    o_ref[...] = (jnp.dot(t, w2_ref[...], preferred_element_type=_f32)
                  + b2_ref[...] + hp_ref[...])


def _mm3(y, stats, gamma, beta, w2, b2, hprev):
    h2 = 2 * HZ
    return pl.pallas_call(
        _mm3_body,
        grid=(GM,),
        in_specs=[
            pl.BlockSpec((MB, h2), lambda m: (m, 0)),
            pl.BlockSpec((8, h2), lambda m: (0, 0)),
            pl.BlockSpec((1, h2), lambda m: (0, 0)),
            pl.BlockSpec((1, h2), lambda m: (0, 0)),
            pl.BlockSpec((h2, HZ), lambda m: (0, 0)),
            pl.BlockSpec((1, HZ), lambda m: (0, 0)),
            pl.BlockSpec((MB, HZ), lambda m: (m, 0)),
        ],
        out_specs=pl.BlockSpec((MB, HZ), lambda m: (m, 0)),
        out_shape=jax.ShapeDtypeStruct((N, HZ), _f32),
    )(y, stats, gamma.reshape(1, h2), beta.reshape(1, h2), w2,
      b2.reshape(1, HZ), hprev)


def _sc_pass_a_body(src_hbm, dst_hbm, eat_hbm, vel_hbm, zs_hbm, zd_hbm,
                    ex_hbm, zs_v, zd_v, vel_v, src_v, dst_v, eat_v, ex_v):
    w = lax.axis_index("s") * NSC + lax.axis_index("c")
    base = w * A_EPT
    pltpu.sync_copy(zs_hbm, zs_v)
    pltpu.sync_copy(zd_hbm, zd_v)
    pltpu.sync_copy(vel_hbm, vel_v)
    velv = vel_v[pl.ds(0, 16)]

    def chunk(ci, carry):
        off = base + ci * A_CH
        pltpu.sync_copy(src_hbm.at[pl.ds(off, A_CH)], src_v)
        pltpu.sync_copy(dst_hbm.at[pl.ds(off, A_CH)], dst_v)
        pltpu.sync_copy(eat_hbm.at[:, pl.ds(off, A_CH)], eat_v)

        def vec(j, c2):
            sl = pl.ds(j * 16, 16)
            si = src_v[sl]
            di = dst_v[sl]
            t = plsc.load_gather(zs_v, [si]) + plsc.load_gather(zd_v, [di])
            for k in range(16):
                t = t + velv[k] * eat_v[k, sl]
            t = jnp.maximum(t, 0.2 * t)
            ex_v[sl] = jnp.exp(t)
            return c2

        lax.fori_loop(0, A_CH // 16, vec, 0)
        pltpu.sync_copy(ex_v, ex_hbm.at[pl.ds(off, A_CH)])
        return carry

    lax.fori_loop(0, A_NCH, chunk, 0)


def _scale_rows(rows_ref, ex_ref):
    @plsc.parallel_loop(0, B_K, 1, unroll=8)
    def _(e):
        s16 = plsc.load_gather(ex_ref, [jnp.full((16,), e, jnp.int32)])
        for f in range(CHW // 16):
            sl = pl.ds(f * 16, 16)
            rows_ref[e, sl] = rows_ref[e, sl] * s16


def _sc_pass_b_body(idx4_hbm, dst_hbm, ex_hbm, z_hbm, zc_hbm, zd_hbm,
                    hc_hbm, den_hbm, acc_sh, den_sh,
                    idxA0, idxA1, dstb0, dstb1, exb0, exb1,
                    rows0, rows1, denr0, denr1,
                    gsem0, gsem1, psA0, psA1, psB0, psB1,
                    ssem0, ssem1, dsem0, dsem1):
    core = lax.axis_index("c")
    tid = lax.axis_index("s")
    ebase = tid * B_EPT
    rbase = tid * ROWS_PT
    idxA = [idxA0, idxA1]
    dstb = [dstb0, dstb1]
    exb = [exb0, exb1]
    rows = [rows0, rows1]
    denr = [denr0, denr1]
    gsem = [gsem0, gsem1]
    psA = [psA0, psA1]
    psB = [psB0, psB1]
    ssem = [ssem0, ssem1]
    dsem = [dsem0, dsem1]
    lane = lax.iota(jnp.int32, 16)
    zcol = jnp.zeros((16,), jnp.int32)

    def wait_rows(s, sem):
        pltpu.make_async_copy(z_hbm.at[pl.ds(0, B_K)], rows[s], sem).wait()

    def wait_idx(s, sem):
        pltpu.make_async_copy(
            idx4_hbm.at[0, pl.ds(0, B_K)], idxA[s], sem).wait()

    def wait_de(s, sem):
        pltpu.make_async_copy(dst_hbm.at[pl.ds(0, B_K)], dstb[s], sem).wait()
        pltpu.make_async_copy(ex_hbm.at[pl.ds(0, B_K)], exb[s], sem).wait()

    def wait_den(s, sem):
        pltpu.make_async_copy(den_hbm.at[pl.ds(0, B_K)], denr[s], sem).wait()

    for cc in range(2):
        c = core * 2 + cc
        c_n = c * N

        pltpu.sync_copy(zc_hbm.at[pl.ds(rbase, ROWS_PT)],
                        acc_sh.at[pl.ds(rbase, ROWS_PT)])
        if cc == 0:

            @pl.when(core == 0)
            def _():
                pltpu.sync_copy(zd_hbm.at[pl.ds(rbase, ROWS_PT)],
                                den_sh.at[pl.ds(rbase, ROWS_PT)])
                pltpu.sync_copy(zd_hbm.at[pl.ds(0, B_K)], denr0)
                pltpu.sync_copy(zd_hbm.at[pl.ds(0, B_K)], denr1)

        plsc.subcore_barrier()

        def den_issue(b, s):
            @pl.when(core == 0)
            def _():
                for jj in range(B_K // 16):
                    sl = pl.ds(jj * 16, 16)
                    plsc.store_scatter(denr[s], [lane + jj * 16, zcol],
                                       exb[s][sl])
                pltpu.async_copy(denr[s], den_sh.at[dstb[s]], dsem[s],
                                 add=True)

        pltpu.sync_copy(idx4_hbm.at[c, pl.ds(ebase, B_K)], idxA0)
        pltpu.sync_copy(dst_hbm.at[pl.ds(ebase, B_K)], dstb0)
        pltpu.sync_copy(ex_hbm.at[pl.ds(ebase, B_K)], exb0)
        pltpu.async_copy(z_hbm.at[idxA0], rows0, gsem0)
        pltpu.async_copy(idx4_hbm.at[c, pl.ds(ebase + B_K, B_K)], idxA1,
                         psA1)
        pltpu.async_copy(dst_hbm.at[pl.ds(ebase + B_K, B_K)], dstb1, psB1)
        pltpu.async_copy(ex_hbm.at[pl.ds(ebase + B_K, B_K)], exb1, psB1)

        wait_rows(0, gsem0)
        wait_idx(1, psA1)
        pltpu.async_copy(z_hbm.at[idxA1], rows1, gsem1)
        pltpu.async_copy(idx4_hbm.at[c, pl.ds(ebase + 2 * B_K, B_K)],
                         idxA0, psA0)
        _scale_rows(rows0, exb0)
        pltpu.async_copy(rows0, acc_sh.at[dstb0], ssem0, add=True)
        if cc == 0:
            den_issue(0, 0)

        def half(b, s):
            o = 1 - s
            wait_rows(s, gsem[s])
            wait_idx(o, psA[o])
            wait_rows(o, ssem[o])
            if cc == 0:

                @pl.when(core == 0)
                def _():
                    wait_den(o, dsem[o])

            off1 = ebase + (b + 1) * B_K
            pltpu.async_copy(dst_hbm.at[pl.ds(off1, B_K)], dstb[o], psB[o])
            pltpu.async_copy(ex_hbm.at[pl.ds(off1, B_K)], exb[o], psB[o])
            pltpu.async_copy(z_hbm.at[idxA[o]], rows[o], gsem[o])
            off2 = ebase + (b + 2) * B_K
            pltpu.async_copy(idx4_hbm.at[c, pl.ds(off2, B_K)], idxA[s],
                             psA[s])
            wait_de(s, psB[s])
            _scale_rows(rows[s], exb[s])
            pltpu.async_copy(rows[s], acc_sh.at[dstb[s]], ssem[s], add=True)
            if cc == 0:
                den_issue(b, s)

        def pair(j, carry):
            half(2 * j + 1, 1)
            half(2 * j + 2, 0)
            return carry

        lax.fori_loop(0, (B_NB - 1) // 2, pair, 0)

        wait_rows(1, gsem1)
        wait_idx(0, psA0)
        wait_de(1, psB1)
        wait_rows(0, ssem0)
        if cc == 0:

            @pl.when(core == 0)
            def _():
                wait_den(0, dsem0)

        plsc.subcore_barrier()

        pltpu.sync_copy(acc_sh.at[pl.ds(rbase, ROWS_PT)],
                        hc_hbm.at[pl.ds(c_n + rbase, ROWS_PT)])
        if cc == 0:

            @pl.when(core == 0)
            def _():
                pltpu.sync_copy(den_sh.at[pl.ds(rbase, ROWS_PT)],
                                den_hbm.at[pl.ds(rbase, ROWS_PT)])

        plsc.subcore_barrier()


@functools.lru_cache(maxsize=None)
def _sc_kernels():
    mesh = plsc.VectorSubcoreMesh(core_axis_name="c", subcore_axis_name="s",
                                  num_cores=NSC, num_subcores=NTS)
    pass_a = pl.kernel(
        _sc_pass_a_body,
        out_type=jax.ShapeDtypeStruct((E_PAD,), _f32),
        mesh=mesh,
        compiler_params=pltpu.CompilerParams(needs_layout_passes=False, use_tc_tiling_on_sc=False),
        scratch_types=[
            pltpu.VMEM((N,), _f32),
            pltpu.VMEM((N,), _f32),
            pltpu.VMEM((16,), _f32),
            pltpu.VMEM((A_CH,), jnp.int32),
            pltpu.VMEM((A_CH,), jnp.int32),
            pltpu.VMEM((16, A_CH), _f32),
            pltpu.VMEM((A_CH,), _f32),
        ],
    )
    pass_b = pl.kernel(
        _sc_pass_b_body,
        out_type=[
            jax.ShapeDtypeStruct((NCH * N, CHW), _f32),
            jax.ShapeDtypeStruct((N, 16), _f32),
        ],
        mesh=mesh,
        compiler_params=pltpu.CompilerParams(needs_layout_passes=False, use_tc_tiling_on_sc=False),
        scratch_types=[
            pltpu.VMEM_SHARED((N, CHW), _f32),
            pltpu.VMEM_SHARED((N, 16), _f32),
            pltpu.VMEM((B_K,), jnp.int32),
            pltpu.VMEM((B_K,), jnp.int32),
            pltpu.VMEM((B_K,), jnp.int32),
            pltpu.VMEM((B_K,), jnp.int32),
            pltpu.VMEM((B_K,), _f32),
            pltpu.VMEM((B_K,), _f32),
            pltpu.VMEM((B_K, CHW), _f32),
            pltpu.VMEM((B_K, CHW), _f32),
            pltpu.VMEM((B_K, 16), _f32),
            pltpu.VMEM((B_K, 16), _f32),
            pltpu.SemaphoreType.DMA,
            pltpu.SemaphoreType.DMA,
            pltpu.SemaphoreType.DMA,
            pltpu.SemaphoreType.DMA,
            pltpu.SemaphoreType.DMA,
            pltpu.SemaphoreType.DMA,
            pltpu.SemaphoreType.DMA,
            pltpu.SemaphoreType.DMA,
            pltpu.SemaphoreType.DMA,
            pltpu.SemaphoreType.DMA,
        ],
    )
    return pass_a, pass_b


def kernel(x, edge_index, edge_attr, Wnm, bnm, Wem, bem, Wc, bc, a_src,
           a_dst, a_edge, W1, b1, gamma, beta, W2, b2, Wout, bout):
    src = edge_index[0]
    dst = edge_index[1]

    vel = Wem @ a_edge.T
    cel = bem @ a_edge.T
    u_s = jnp.einsum("lij,lj->li", Wc, a_src)
    u_d = jnp.einsum("lij,lj->li", Wc, a_dst)
    c_s = jnp.einsum("lj,lj->l", bc, a_src)
    c_d = jnp.einsum("lj,lj->l", bc, a_dst) + cel

    pad = E_PAD - E
    src_p = jnp.concatenate([src, jnp.zeros((pad,), jnp.int32)])
    dst_p = jnp.concatenate([dst, jnp.zeros((pad,), jnp.int32)])
    eat_p = jnp.concatenate(
        [edge_attr.T, jnp.zeros((16, pad), _f32)], axis=1)
    idx4 = src_p[None, :] + (jnp.arange(NCH, dtype=jnp.int32) * N)[:, None]
    zc = jnp.zeros((N, CHW), _f32)
    zd16 = jnp.zeros((N, 16), _f32)

    h = _mm_bias(x, Wnm, bnm)

    for i in range(NL):
        u_pad = jnp.zeros((HZ, 128), _f32)
        u_pad = u_pad.at[:, 0].set(u_s[i]).at[:, 1].set(u_d[i])
        cu_pad = jnp.zeros((1, 128), _f32)
        cu_pad = cu_pad.at[0, 0].set(c_s[i]).at[0, 1].set(c_d[i])
        szd = _mmszd(h, u_pad, cu_pad)
        zs = szd[:, 0]
        zd = szd[:, 1]

        pass_a, pass_b = _sc_kernels()
        ex_pad = pass_a(src_p, dst_p, eat_p, vel[:, i], zs, zd)
        z_ch = _mmz(h, Wc[i], bc[i])

        hconv_un, den2d = pass_b(idx4, dst_p, ex_pad, z_ch, zc, zd16)

        y, stats = _mm2(hconv_un, den2d, W1[i], b1[i])
        h = _mm3(y, stats, gamma[i], beta[i], W2[i], b2[i], h)

    wout_pad = jnp.zeros((HZ, 128), _f32).at[:, :64].set(Wout)
    bout_pad = jnp.zeros((128,), _f32).at[:64].set(bout)
    out = _mm_bias(h, wout_pad, bout_pad)
    return out[:, :64]


# speedup vs baseline: 1.0327x; 1.0327x over previous
import functools

import jax
import jax.numpy as jnp
from jax import lax
from jax.experimental import pallas as pl
from jax.experimental.pallas import tpu as pltpu
from jax.experimental.pallas import tpu_sc as plsc

N = 10000
E = 160000
HZ = 512
NL = 3

NCH = 4
CHW = 128
NSC = 2
NTS = 16
NW = NSC * NTS

A_EPT = 5120
E_PAD = NW * A_EPT
A_CH = 1024
A_NCH = A_EPT // A_CH

B_EPT = E // NTS
B_K = 80
B_NB = B_EPT // B_K
ROWS_PT = N // NTS
RP = 125

MB = 1000
GM = N // MB

_f32 = jnp.float32


def _mm_bias_body(x_ref, w_ref, b_ref, o_ref):
    o_ref[...] = jnp.dot(x_ref[...], w_ref[...],
                         preferred_element_type=_f32) + b_ref[...]


def _mm_bias(x, w, b, mb=MB):
    m, k = x.shape
    n = w.shape[1]
    return pl.pallas_call(
        _mm_bias_body,
        grid=(m // mb,),
        in_specs=[
            pl.BlockSpec((mb, k), lambda i: (i, 0)),
            pl.BlockSpec((k, n), lambda i: (0, 0)),
            pl.BlockSpec((1, n), lambda i: (0, 0)),
        ],
        out_specs=pl.BlockSpec((mb, n), lambda i: (i, 0)),
        out_shape=jax.ShapeDtypeStruct((m, n), _f32),
    )(x, w, b.reshape(1, n))


def _mmz_body(h_ref, wc_ref, bc_ref, z_ref):
    z_ref[...] = jnp.dot(h_ref[...], wc_ref[...],
                         preferred_element_type=_f32) + bc_ref[...]


def _mmz(h, wc, bc):
    return pl.pallas_call(
        _mmz_body,
        grid=(GM, NCH),
        in_specs=[
            pl.BlockSpec((MB, HZ), lambda m, c: (m, 0)),
            pl.BlockSpec((HZ, CHW), lambda m, c: (0, c)),
            pl.BlockSpec((1, CHW), lambda m, c: (0, c)),
        ],
        out_specs=pl.BlockSpec((MB, CHW), lambda m, c: (c * GM + m, 0)),
        out_shape=jax.ShapeDtypeStruct((NCH * N, CHW), _f32),
    )(h, wc, bc.reshape(1, HZ))


def _mmszd_body(h_ref, u_ref, cu_ref, szd_ref):
    szd_ref[...] = jnp.dot(h_ref[...], u_ref[...],
                           preferred_element_type=_f32) + cu_ref[...]


def _mmszd(h, u_pad, cu_pad):
    return pl.pallas_call(
        _mmszd_body,
        grid=(GM,),
        in_specs=[
            pl.BlockSpec((MB, HZ), lambda m: (m, 0)),
            pl.BlockSpec((HZ, 128), lambda m: (0, 0)),
            pl.BlockSpec((1, 128), lambda m: (0, 0)),
        ],
        out_specs=pl.BlockSpec((MB, 128), lambda m: (m, 0)),
        out_shape=jax.ShapeDtypeStruct((N, 128), _f32),
    )(h, u_pad, cu_pad)


def _mm2a_body(hc_ref, den_ref, w1_ref, b1_ref, y_ref):
    kc = pl.program_id(1)
    recip = 1.0 / (den_ref[:, 0:1] + 1e-16)
    a = hc_ref[...] * recip
    part = jnp.dot(a, w1_ref[...], preferred_element_type=_f32)

    @pl.when(kc == 0)
    def _():
        y_ref[...] = part + b1_ref[...]

    @pl.when(kc == 1)
    def _():
        y_ref[...] += part


def _mm2a(hc02, den2d, w1_02, b1):
    h2 = 2 * HZ
    return pl.pallas_call(
        _mm2a_body,
        grid=(GM, 2),
        in_specs=[
            pl.BlockSpec((MB, CHW), lambda m, kc: (kc * GM + m, 0)),
            pl.BlockSpec((MB, 16), lambda m, kc: (m, 0)),
            pl.BlockSpec((CHW, h2), lambda m, kc: (kc, 0)),
            pl.BlockSpec((1, h2), lambda m, kc: (0, 0)),
        ],
        out_specs=pl.BlockSpec((MB, h2), lambda m, kc: (m, 0)),
        out_shape=jax.ShapeDtypeStruct((N, h2), _f32),
    )(hc02, den2d, w1_02, b1.reshape(1, h2))


def _mm2b_body(hc_ref, den_ref, yin_ref, w1_ref, y_ref, st_ref):
    m = pl.program_id(0)
    kc = pl.program_id(1)
    recip = 1.0 / (den_ref[:, 0:1] + 1e-16)
    a = hc_ref[...] * recip
    part = jnp.dot(a, w1_ref[...], preferred_element_type=_f32)

    @pl.when(kc == 0)
    def _():
        y_ref[...] = yin_ref[...] + part

    @pl.when(kc == 1)
    def _():
        y = y_ref[...] + part
        y_ref[...] = y
        s = jnp.sum(y, axis=0, keepdims=True)
        s2 = jnp.sum(y * y, axis=0, keepdims=True)
        st = jnp.concatenate([s, s2, jnp.zeros((6, y.shape[1]), _f32)], axis=0)

        @pl.when(m == 0)
        def _():
            st_ref[...] = st

        @pl.when(m > 0)
        def _():
            st_ref[...] += st


def _mm2b(hc13, den2d, y_part, w1_13):
    h2 = 2 * HZ
    return pl.pallas_call(
        _mm2b_body,
        grid=(GM, 2),
        in_specs=[
            pl.BlockSpec((MB, CHW), lambda m, kc: (kc * GM + m, 0)),
            pl.BlockSpec((MB, 16), lambda m, kc: (m, 0)),
            pl.BlockSpec((MB, h2), lambda m, kc: (m, 0)),
            pl.BlockSpec((CHW, h2), lambda m, kc: (kc, 0)),
        ],
        out_specs=[
            pl.BlockSpec((MB, h2), lambda m, kc: (m, 0)),
            pl.BlockSpec((8, h2), lambda m, kc: (0, 0)),
        ],
        out_shape=[
            jax.ShapeDtypeStruct((N, h2), _f32),
            jax.ShapeDtypeStruct((8, h2), _f32),
        ],
    )(hc13, den2d, y_part, w1_13)


def _mm3_body(y_ref, st_ref, g_ref, be_ref, w2_ref, b2_ref, hp_ref, o_ref):
    mu = st_ref[0:1, :] * (1.0 / N)
    var = st_ref[1:2, :] * (1.0 / N) - mu * mu
    aa = g_ref[...] * lax.rsqrt(var + 1e-5)
    bb = be_ref[...] - mu * aa
    t = y_ref[...] * aa + bb
    t = jnp.maximum(t, 0.01 * t)
    o_ref[...] = (jnp.dot(t, w2_ref[...], preferred_element_type=_f32)
                  + b2_ref[...] + hp_ref[...])


def _mm3(y, stats, gamma, beta, w2, b2, hprev):
    h2 = 2 * HZ
    return pl.pallas_call(
        _mm3_body,
        grid=(GM,),
        in_specs=[
            pl.BlockSpec((MB, h2), lambda m: (m, 0)),
            pl.BlockSpec((8, h2), lambda m: (0, 0)),
            pl.BlockSpec((1, h2), lambda m: (0, 0)),
            pl.BlockSpec((1, h2), lambda m: (0, 0)),
            pl.BlockSpec((h2, HZ), lambda m: (0, 0)),
            pl.BlockSpec((1, HZ), lambda m: (0, 0)),
            pl.BlockSpec((MB, HZ), lambda m: (m, 0)),
        ],
        out_specs=pl.BlockSpec((MB, HZ), lambda m: (m, 0)),
        out_shape=jax.ShapeDtypeStruct((N, HZ), _f32),
    )(y, stats, gamma.reshape(1, h2), beta.reshape(1, h2), w2,
      b2.reshape(1, HZ), hprev)


def _sc_pass_a_body(src_hbm, dst_hbm, eat_hbm, vel_hbm, zs_hbm, zd_hbm,
                    ex_hbm, zs_v, zd_v, vel_v, src_v, dst_v, eat_v, ex_v):
    w = lax.axis_index("s") * NSC + lax.axis_index("c")
    base = w * A_EPT
    pltpu.sync_copy(zs_hbm, zs_v)
    pltpu.sync_copy(zd_hbm, zd_v)
    pltpu.sync_copy(vel_hbm, vel_v)
    velv = vel_v[pl.ds(0, 16)]

    def chunk(ci, carry):
        off = base + ci * A_CH
        pltpu.sync_copy(src_hbm.at[pl.ds(off, A_CH)], src_v)
        pltpu.sync_copy(dst_hbm.at[pl.ds(off, A_CH)], dst_v)
        pltpu.sync_copy(eat_hbm.at[:, pl.ds(off, A_CH)], eat_v)

        def vec(j, c2):
            sl = pl.ds(j * 16, 16)
            si = src_v[sl]
            di = dst_v[sl]
            t = plsc.load_gather(zs_v, [si]) + plsc.load_gather(zd_v, [di])
            for k in range(16):
                t = t + velv[k] * eat_v[k, sl]
            t = jnp.maximum(t, 0.2 * t)
            ex_v[sl] = jnp.exp(t)
            return c2

        lax.fori_loop(0, A_CH // 16, vec, 0)
        pltpu.sync_copy(ex_v, ex_hbm.at[pl.ds(off, A_CH)])
        return carry

    lax.fori_loop(0, A_NCH, chunk, 0)


def _scale_rows(rows_ref, ex_ref):
    @plsc.parallel_loop(0, B_K, 1, unroll=4)
    def _(e):
        s16 = plsc.load_gather(ex_ref, [jnp.full((16,), e, jnp.int32)])
        for f in range(CHW // 16):
            sl = pl.ds(f * 16, 16)
            rows_ref[e, sl] = rows_ref[e, sl] * s16


def _pass_b_core(cc, idx4_hbm, dst_hbm, ex_hbm, z_hbm, zc_hbm, zd_hbm,
                 hc_hbm, den_hbm, acc_sh, den_sh,
                 idxA0, idxA1, dstb0, dstb1, exb0, exb1,
                 rows0, rows1, denr0, denr1,
                 gsem0, gsem1, psA0, psA1, psB0, psB1,
                 ssem0, ssem1, dsem0, dsem1):
    core = lax.axis_index("c")
    tid = lax.axis_index("s")
    ebase = tid * B_EPT
    rbase = tid * ROWS_PT
    idxA = [idxA0, idxA1]
    dstb = [dstb0, dstb1]
    exb = [exb0, exb1]
    rows = [rows0, rows1]
    denr = [denr0, denr1]
    gsem = [gsem0, gsem1]
    psA = [psA0, psA1]
    psB = [psB0, psB1]
    ssem = [ssem0, ssem1]
    dsem = [dsem0, dsem1]
    lane = lax.iota(jnp.int32, 16)
    zcol = jnp.zeros((16,), jnp.int32)

    def wait_rows(s, sem):
        pltpu.make_async_copy(z_hbm.at[pl.ds(0, B_K)], rows[s], sem).wait()

    def wait_idx(s, sem):
        pltpu.make_async_copy(
            idx4_hbm.at[0, pl.ds(0, B_K)], idxA[s], sem).wait()

    def wait_de(s, sem):
        pltpu.make_async_copy(dst_hbm.at[pl.ds(0, B_K)], dstb[s], sem).wait()
        pltpu.make_async_copy(ex_hbm.at[pl.ds(0, B_K)], exb[s], sem).wait()

    def wait_den(s, sem):
        pltpu.make_async_copy(den_hbm.at[pl.ds(0, B_K)], denr[s], sem).wait()

    c = core * 2 + cc

    pltpu.sync_copy(zc_hbm.at[pl.ds(rbase, ROWS_PT)],
                    acc_sh.at[pl.ds(rbase, ROWS_PT)])
    if cc == 0:

        @pl.when(core == 0)
        def _():
            pltpu.sync_copy(zd_hbm.at[pl.ds(rbase, ROWS_PT)],
                            den_sh.at[pl.ds(rbase, ROWS_PT)])
            pltpu.sync_copy(zd_hbm.at[pl.ds(0, B_K)], denr0)
            pltpu.sync_copy(zd_hbm.at[pl.ds(0, B_K)], denr1)

    plsc.subcore_barrier()

    def den_issue(s):
        @pl.when(core == 0)
        def _():
            for jj in range(B_K // 16):
                sl = pl.ds(jj * 16, 16)
                plsc.store_scatter(denr[s], [lane + jj * 16, zcol],
                                   exb[s][sl])
            pltpu.async_copy(denr[s], den_sh.at[dstb[s]], dsem[s],
                             add=True)

    pltpu.sync_copy(idx4_hbm.at[c, pl.ds(ebase, B_K)], idxA0)
    pltpu.sync_copy(dst_hbm.at[pl.ds(ebase, B_K)], dstb0)
    pltpu.sync_copy(ex_hbm.at[pl.ds(ebase, B_K)], exb0)
    pltpu.async_copy(z_hbm.at[idxA0], rows0, gsem0)
    pltpu.async_copy(idx4_hbm.at[c, pl.ds(ebase + B_K, B_K)], idxA1,
                     psA1)
    pltpu.async_copy(dst_hbm.at[pl.ds(ebase + B_K, B_K)], dstb1, psB1)
    pltpu.async_copy(ex_hbm.at[pl.ds(ebase + B_K, B_K)], exb1, psB1)

    wait_rows(0, gsem0)
    wait_idx(1, psA1)
    pltpu.async_copy(z_hbm.at[idxA1], rows1, gsem1)
    pltpu.async_copy(idx4_hbm.at[c, pl.ds(ebase + 2 * B_K, B_K)],
                     idxA0, psA0)
    _scale_rows(rows0, exb0)
    pltpu.async_copy(rows0, acc_sh.at[dstb0], ssem0, add=True)
    if cc == 0:
        den_issue(0)

    def half(b, s):
        o = 1 - s
        wait_rows(s, gsem[s])
        wait_idx(o, psA[o])
        wait_rows(o, ssem[o])
        if cc == 0:

            @pl.when(core == 0)
            def _():
                wait_den(o, dsem[o])

        off1 = ebase + (b + 1) * B_K
        pltpu.async_copy(dst_hbm.at[pl.ds(off1, B_K)], dstb[o], psB[o])
        pltpu.async_copy(ex_hbm.at[pl.ds(off1, B_K)], exb[o], psB[o])
        pltpu.async_copy(z_hbm.at[idxA[o]], rows[o], gsem[o])
        off2 = ebase + (b + 2) * B_K
        pltpu.async_copy(idx4_hbm.at[c, pl.ds(off2, B_K)], idxA[s],
                         psA[s])
        wait_de(s, psB[s])
        _scale_rows(rows[s], exb[s])
        pltpu.async_copy(rows[s], acc_sh.at[dstb[s]], ssem[s], add=True)
        if cc == 0:
            den_issue(s)

    def pair(j, carry):
        half(2 * j + 1, 1)
        half(2 * j + 2, 0)
        return carry

    lax.fori_loop(0, (B_NB - 1) // 2, pair, 0)

    wait_rows(1, gsem1)
    wait_idx(0, psA0)
    wait_de(1, psB1)
    wait_rows(0, ssem0)
    if cc == 0:

        @pl.when(core == 0)
        def _():
            wait_den(0, dsem0)

    plsc.subcore_barrier()

    pltpu.sync_copy(acc_sh.at[pl.ds(rbase, ROWS_PT)],
                    hc_hbm.at[pl.ds(core * N + rbase, ROWS_PT)])
    if cc == 0:

        @pl.when(core == 0)
        def _():
            pltpu.sync_copy(den_sh.at[pl.ds(rbase, ROWS_PT)],
                            den_hbm.at[pl.ds(rbase, ROWS_PT)])

    plsc.subcore_barrier()


def _pass_b0_body(idx4_hbm, dst_hbm, ex_hbm, z_hbm, zc_hbm, zd_hbm,
                  hc_hbm, den_hbm, acc_sh, den_sh, *bufs):
    _pass_b_core(0, idx4_hbm, dst_hbm, ex_hbm, z_hbm, zc_hbm, zd_hbm,
                 hc_hbm, den_hbm, acc_sh, den_sh, *bufs)


def _pass_b1_body(idx4_hbm, dst_hbm, ex_hbm, z_hbm, zc_hbm, zd_hbm,
                  hc_hbm, acc_sh, den_sh, *bufs):
    _pass_b_core(1, idx4_hbm, dst_hbm, ex_hbm, z_hbm, zc_hbm, zd_hbm,
                 hc_hbm, None, acc_sh, den_sh, *bufs)


@functools.lru_cache(maxsize=None)
def _sc_kernels():
    mesh = plsc.VectorSubcoreMesh(core_axis_name="c", subcore_axis_name="s",
                                  num_cores=NSC, num_subcores=NTS)
    pass_a = pl.kernel(
        _sc_pass_a_body,
        out_type=jax.ShapeDtypeStruct((E_PAD,), _f32),
        mesh=mesh,
        compiler_params=pltpu.CompilerParams(needs_layout_passes=False, use_tc_tiling_on_sc=False),
        scratch_types=[
            pltpu.VMEM((N,), _f32),
            pltpu.VMEM((N,), _f32),
            pltpu.VMEM((16,), _f32),
            pltpu.VMEM((A_CH,), jnp.int32),
            pltpu.VMEM((A_CH,), jnp.int32),
            pltpu.VMEM((16, A_CH), _f32),
            pltpu.VMEM((A_CH,), _f32),
        ],
    )
    scb = [
            pltpu.VMEM_SHARED((N, CHW), _f32),
            pltpu.VMEM_SHARED((N, 16), _f32),
            pltpu.VMEM((B_K,), jnp.int32),
            pltpu.VMEM((B_K,), jnp.int32),
            pltpu.VMEM((B_K,), jnp.int32),
            pltpu.VMEM((B_K,), jnp.int32),
            pltpu.VMEM((B_K,), _f32),
            pltpu.VMEM((B_K,), _f32),
            pltpu.VMEM((B_K, CHW), _f32),
            pltpu.VMEM((B_K, CHW), _f32),
            pltpu.VMEM((B_K, 16), _f32),
            pltpu.VMEM((B_K, 16), _f32),
            pltpu.SemaphoreType.DMA,
            pltpu.SemaphoreType.DMA,
            pltpu.SemaphoreType.DMA,
            pltpu.SemaphoreType.DMA,
            pltpu.SemaphoreType.DMA,
            pltpu.SemaphoreType.DMA,
            pltpu.SemaphoreType.DMA,
            pltpu.SemaphoreType.DMA,
            pltpu.SemaphoreType.DMA,
            pltpu.SemaphoreType.DMA,
    ]
    cp = pltpu.CompilerParams(needs_layout_passes=False,
                              use_tc_tiling_on_sc=False)
    pass_b0 = pl.kernel(
        _pass_b0_body,
        out_type=[
            jax.ShapeDtypeStruct((NSC * N, CHW), _f32),
            jax.ShapeDtypeStruct((N, 16), _f32),
        ],
        mesh=mesh, compiler_params=cp, scratch_types=scb,
    )
    pass_b1 = pl.kernel(
        _pass_b1_body,
        out_type=jax.ShapeDtypeStruct((NSC * N, CHW), _f32),
        mesh=mesh, compiler_params=cp, scratch_types=scb,
    )
    return pass_a, pass_b0, pass_b1


def kernel(x, edge_index, edge_attr, Wnm, bnm, Wem, bem, Wc, bc, a_src,
           a_dst, a_edge, W1, b1, gamma, beta, W2, b2, Wout, bout):
    src = edge_index[0]
    dst = edge_index[1]

    vel = Wem @ a_edge.T
    cel = bem @ a_edge.T
    u_s = jnp.einsum("lij,lj->li", Wc, a_src)
    u_d = jnp.einsum("lij,lj->li", Wc, a_dst)
    c_s = jnp.einsum("lj,lj->l", bc, a_src)
    c_d = jnp.einsum("lj,lj->l", bc, a_dst) + cel

    pad = E_PAD - E
    src_p = jnp.concatenate([src, jnp.zeros((pad,), jnp.int32)])
    dst_p = jnp.concatenate([dst, jnp.zeros((pad,), jnp.int32)])
    eat_p = jnp.concatenate(
        [edge_attr.T, jnp.zeros((16, pad), _f32)], axis=1)
    idx4 = src_p[None, :] + (jnp.arange(NCH, dtype=jnp.int32) * N)[:, None]
    zc = jnp.zeros((N, CHW), _f32)
    zd16 = jnp.zeros((N, 16), _f32)

    h = _mm_bias(x, Wnm, bnm)

    pass_a, pass_b0, pass_b1 = _sc_kernels()
    for i in range(NL):
        u_pad = jnp.zeros((HZ, 128), _f32)
        u_pad = u_pad.at[:, 0].set(u_s[i]).at[:, 1].set(u_d[i])
        cu_pad = jnp.zeros((1, 128), _f32)
        cu_pad = cu_pad.at[0, 0].set(c_s[i]).at[0, 1].set(c_d[i])
        szd = _mmszd(h, u_pad, cu_pad)
        zs = szd[:, 0]
        zd = szd[:, 1]

        ex_pad = pass_a(src_p, dst_p, eat_p, vel[:, i], zs, zd)
        z_ch = _mmz(h, Wc[i], bc[i])

        hc02, den2d = pass_b0(idx4, dst_p, ex_pad, z_ch, zc, zd16)
        hc13 = pass_b1(idx4, dst_p, ex_pad, z_ch, zc, zd16)

        w1_02 = jnp.concatenate([W1[i][0:CHW], W1[i][2 * CHW:3 * CHW]], 0)
        w1_13 = jnp.concatenate([W1[i][CHW:2 * CHW], W1[i][3 * CHW:]], 0)
        y_part = _mm2a(hc02, den2d, w1_02, b1[i])
        y, stats = _mm2b(hc13, den2d, y_part, w1_13)
        h = _mm3(y, stats, gamma[i], beta[i], W2[i], b2[i], h)

    wout_pad = jnp.zeros((HZ, 128), _f32).at[:, :64].set(Wout)
    bout_pad = jnp.zeros((128,), _f32).at[:64].set(bout)
    out = _mm_bias(h, wout_pad, bout_pad)
    return out[:, :64]
